# Initial kernel scaffold; baseline (speedup 1.0000x reference)
import jax
import jax.numpy as jnp
from jax.experimental import pallas as pl


def kernel(x_unit_base, hat_t, rel_edge_index, rel_edge_type, a_r_params):
    raise NotImplementedError("write your pallas kernel here")


# speedup vs baseline: 78.9113x; 78.9113x over previous
import functools

import jax
import jax.numpy as jnp
from jax import lax
from jax.experimental import pallas as pl
from jax.experimental.pallas import tpu as pltpu
from jax.experimental.pallas import tpu_sc as plsc

NC = 2
NS = 16
NW = NC * NS
L = 16

N_UNITS = 10000
N_EDGES = 320000
NODE_DIM = 128
N_REL = 16

ROWS = 80
EPT = ROWS * 128
PE = NW * EPT
ACC = 10240
DUMMY = N_UNITS
TBL = 160256
TBL_SLICE = TBL // NS


def _scores_body(x_ref, w_ref, s_ref, c_ref):
    s = jnp.dot(x_ref[...], w_ref[...],
                preferred_element_type=jnp.float32,
                precision=lax.Precision.HIGHEST)
    s_ref[...] = s
    c = jnp.max(s[:, :N_REL]) + jnp.max(s[:, N_REL:])
    c_ref[0, 0] = jnp.maximum(c, 0.2 * c)


def _scores(x, w):
    return pl.pallas_call(
        _scores_body,
        out_shape=[
            jax.ShapeDtypeStruct((N_UNITS, 2 * N_REL), jnp.float32),
            jax.ShapeDtypeStruct((1, 1), jnp.float32),
        ],
        out_specs=[
            pl.BlockSpec(memory_space=pltpu.VMEM),
            pl.BlockSpec(memory_space=pltpu.SMEM),
        ],
    )(x, w)


def _edge_body(sl_hbm, sr_hbm, hat_hbm, src_hbm, dst_hbm, typ_hbm, c_hbm,
               parts_hbm,
               srcb, dstb, typb, ilb, irb, avb, bvb, hatb, accd, accn, cb,
               sl_sh, sr_sh, sem_a, sem_b, sem_c):
    cid = lax.axis_index("c")
    sid = lax.axis_index("s")
    wid = sid * NC + cid
    row0 = wid * ROWS

    off = sid * TBL_SLICE
    pltpu.async_copy(sl_hbm.at[pl.ds(off, TBL_SLICE)],
                     accd.at[pl.ds(0, TBL_SLICE)], sem_a)
    pltpu.async_copy(sr_hbm.at[pl.ds(off, TBL_SLICE)],
                     accn.at[pl.ds(0, TBL_SLICE)], sem_a)
    pltpu.async_copy(hat_hbm, hatb, sem_b)
    pltpu.async_copy(c_hbm, cb, sem_b)
    pltpu.async_copy(src_hbm.at[pl.ds(row0, ROWS)], srcb, sem_c)
    pltpu.async_copy(dst_hbm.at[pl.ds(row0, ROWS)], dstb, sem_c)
    pltpu.async_copy(typ_hbm.at[pl.ds(row0, ROWS)], typb, sem_c)

    pltpu.make_async_copy(sl_hbm.at[pl.ds(off, TBL_SLICE)],
                          accd.at[pl.ds(0, TBL_SLICE)], sem_a).wait()
    pltpu.make_async_copy(sr_hbm.at[pl.ds(off, TBL_SLICE)],
                          accn.at[pl.ds(0, TBL_SLICE)], sem_a).wait()
    pltpu.async_copy(accd.at[pl.ds(0, TBL_SLICE)],
                     sl_sh.at[pl.ds(off, TBL_SLICE)], sem_a)
    pltpu.async_copy(accn.at[pl.ds(0, TBL_SLICE)],
                     sr_sh.at[pl.ds(off, TBL_SLICE)], sem_a)
    pltpu.make_async_copy(accd.at[pl.ds(0, TBL_SLICE)],
                          sl_sh.at[pl.ds(off, TBL_SLICE)], sem_a).wait()
    pltpu.make_async_copy(accn.at[pl.ds(0, TBL_SLICE)],
                          sr_sh.at[pl.ds(off, TBL_SLICE)], sem_a).wait()

    pltpu.make_async_copy(src_hbm.at[pl.ds(row0, ROWS)], srcb, sem_c).wait()
    pltpu.make_async_copy(dst_hbm.at[pl.ds(row0, ROWS)], dstb, sem_c).wait()
    pltpu.make_async_copy(typ_hbm.at[pl.ds(row0, ROWS)], typb, sem_c).wait()

    zero = jnp.zeros((L,), jnp.float32)

    def zbody(i, _):
        accd[pl.ds(i * L, L)] = zero
        accn[pl.ds(i * L, L)] = zero
        return 0

    lax.fori_loop(0, ACC // L, zbody, 0)

    def ibody(i, _):
        j = i // 8
        k = (i % 8) * L
        t16 = typb[j, pl.ds(k, L)]
        ilb[j, pl.ds(k, L)] = srcb[j, pl.ds(k, L)] * N_REL + t16
        irb[j, pl.ds(k, L)] = dstb[j, pl.ds(k, L)] * N_REL + t16
        return 0

    lax.fori_loop(0, ROWS * 8, ibody, 0)

    plsc.subcore_barrier()

    def fire(j, _):
        pltpu.async_copy(sl_sh.at[ilb.at[j]], avb.at[j], sem_a)
        pltpu.async_copy(sr_sh.at[irb.at[j]], bvb.at[j], sem_a)
        return 0

    lax.fori_loop(0, ROWS, fire, 0)

    def drain(j, _):
        pltpu.make_async_copy(sl_sh.at[ilb.at[j]], avb.at[j], sem_a).wait()
        pltpu.make_async_copy(sr_sh.at[irb.at[j]], bvb.at[j], sem_a).wait()
        return 0

    lax.fori_loop(0, ROWS, drain, 0)

    pltpu.make_async_copy(hat_hbm, hatb, sem_b).wait()
    pltpu.make_async_copy(c_hbm, cb, sem_b).wait()
    cvec = cb[...]

    def ebody(i, _):
        j = i // 8
        k = (i % 8) * L
        s = avb[j, pl.ds(k, L)] + bvb[j, pl.ds(k, L)]
        e = jnp.maximum(s, 0.2 * s)
        ex = jnp.exp(e - cvec)
        h = plsc.load_gather(hatb, [dstb[j, pl.ds(k, L)]])
        s16 = srcb[j, pl.ds(k, L)]
        plsc.addupdate_scatter(accd, [s16], ex)
        plsc.addupdate_scatter(accn, [s16], ex * h)
        return 0

    lax.fori_loop(0, ROWS * 8, ebody, 0)

    pltpu.sync_copy(accd, parts_hbm.at[pl.ds(wid * 2 * ACC, ACC)])
    pltpu.sync_copy(accn, parts_hbm.at[pl.ds(wid * 2 * ACC + ACC, ACC)])


@functools.cache
def _edge_kernel():
  return functools.partial(
    pl.kernel,
    out_type=jax.ShapeDtypeStruct((NW * 2 * ACC,), jnp.float32),
    mesh=plsc.VectorSubcoreMesh(core_axis_name="c", subcore_axis_name="s",
                                num_cores=NC, num_subcores=NS),
    compiler_params=pltpu.CompilerParams(needs_layout_passes=False),
    scratch_types=[
        pltpu.VMEM((ROWS, 128), jnp.int32),
        pltpu.VMEM((ROWS, 128), jnp.int32),
        pltpu.VMEM((ROWS, 128), jnp.int32),
        pltpu.VMEM((ROWS, 128), jnp.int32),
        pltpu.VMEM((ROWS, 128), jnp.int32),
        pltpu.VMEM((ROWS, 128), jnp.float32),
        pltpu.VMEM((ROWS, 128), jnp.float32),
        pltpu.VMEM((N_UNITS,), jnp.float32),
        pltpu.VMEM((ACC,), jnp.float32),
        pltpu.VMEM((ACC,), jnp.float32),
        pltpu.VMEM((L,), jnp.float32),
        pltpu.VMEM_SHARED((TBL,), jnp.float32),
        pltpu.VMEM_SHARED((TBL,), jnp.float32),
        pltpu.SemaphoreType.DMA,
        pltpu.SemaphoreType.DMA,
        pltpu.SemaphoreType.DMA,
    ],
  )(_edge_body)


def _combine_body(p_ref, o_ref):
    tot = jnp.sum(p_ref[...], axis=0, keepdims=True)
    den = tot[:, :ACC]
    num = tot[:, ACC:]
    o_ref[...] = jnp.where(den != 0.0, num / den, 0.0)


def _combine(parts):
    return pl.pallas_call(
        _combine_body,
        out_shape=jax.ShapeDtypeStruct((1, ACC), jnp.float32),
    )(parts)


def kernel(x_unit_base, hat_t, rel_edge_index, rel_edge_type, a_r_params):
    w = jnp.concatenate([a_r_params[:, :NODE_DIM].T,
                         a_r_params[:, NODE_DIM:].T], axis=1)
    s, c = _scores(x_unit_base, w)

    sl_flat = jnp.pad(s[:, :N_REL].reshape(-1), (0, TBL - N_UNITS * N_REL))
    sr_flat = jnp.pad(s[:, N_REL:].reshape(-1), (0, TBL - N_UNITS * N_REL))
    cvec = jnp.broadcast_to(c.reshape(()), (L,))

    src = rel_edge_index[0].astype(jnp.int32)
    dst = rel_edge_index[1].astype(jnp.int32)
    typ = rel_edge_type.astype(jnp.int32)
    pad = PE - N_EDGES
    src_p = jnp.concatenate(
        [src, jnp.full((pad,), DUMMY, jnp.int32)]).reshape(NW * ROWS, 128)
    dst_p = jnp.concatenate(
        [dst, jnp.zeros((pad,), jnp.int32)]).reshape(NW * ROWS, 128)
    typ_p = jnp.concatenate(
        [typ, jnp.zeros((pad,), jnp.int32)]).reshape(NW * ROWS, 128)

    parts = _edge_kernel()(sl_flat, sr_flat, hat_t, src_p, dst_p, typ_p, cvec)
    g = _combine(parts.reshape(NW, 2 * ACC))
    return g[0, :N_UNITS, None]


# speedup vs baseline: 100.6301x; 1.2752x over previous
import functools

import jax
import jax.numpy as jnp
from jax import lax
from jax.experimental import pallas as pl
from jax.experimental.pallas import tpu as pltpu
from jax.experimental.pallas import tpu_sc as plsc

NC = 2
NS = 16
NW = NC * NS
L = 16

N_UNITS = 10000
N_EDGES = 320000
NODE_DIM = 128
N_REL = 16

EPT = N_EDGES // NW
PADD = 10240
ROWS = PADD // 128
NB = 10
RPB = ROWS // NB
ACC = 10240
DUMMY = N_UNITS
TBL = 160256
TBL_SLICE = TBL // NS


def _scores_body(x_ref, w_ref, s_ref, c_ref):
    s = jnp.dot(x_ref[...], w_ref[...],
                preferred_element_type=jnp.float32,
                precision=lax.Precision.HIGHEST)
    s_ref[...] = s
    c = jnp.max(s[:, :N_REL]) + jnp.max(s[:, N_REL:])
    c_ref[0, 0] = jnp.maximum(c, 0.2 * c)


def _scores(x, w):
    return pl.pallas_call(
        _scores_body,
        out_shape=[
            jax.ShapeDtypeStruct((N_UNITS, 2 * N_REL), jnp.float32),
            jax.ShapeDtypeStruct((1, 1), jnp.float32),
        ],
        out_specs=[
            pl.BlockSpec(memory_space=pltpu.VMEM),
            pl.BlockSpec(memory_space=pltpu.SMEM),
        ],
    )(x, w)


def _edge_body(sl_hbm, sr_hbm, hat_hbm, src_hbm, dst_hbm, typ_hbm, c_hbm,
               parts_hbm,
               srcb, dstb, typb, ilb, irb, avb, bvb, hatb, accd, accn, cb,
               sl_sh, sr_sh, sem_a, sem_b, sem_c, sem_g0, sem_g1):
    cid = lax.axis_index("c")
    sid = lax.axis_index("s")
    wid = sid * NC + cid
    e0 = wid * EPT

    off = sid * TBL_SLICE
    pltpu.async_copy(sl_hbm.at[pl.ds(off, TBL_SLICE)],
                     avb.at[pl.ds(0, TBL_SLICE)], sem_a)
    pltpu.async_copy(sr_hbm.at[pl.ds(off, TBL_SLICE)],
                     bvb.at[pl.ds(0, TBL_SLICE)], sem_a)
    pltpu.async_copy(hat_hbm, hatb, sem_b)
    pltpu.async_copy(c_hbm, cb, sem_b)
    pltpu.async_copy(src_hbm.at[pl.ds(e0, EPT)], srcb.at[pl.ds(0, EPT)], sem_c)
    pltpu.async_copy(dst_hbm.at[pl.ds(e0, EPT)], dstb.at[pl.ds(0, EPT)], sem_c)
    pltpu.async_copy(typ_hbm.at[pl.ds(e0, EPT)], typb.at[pl.ds(0, EPT)], sem_c)

    pltpu.make_async_copy(sl_hbm.at[pl.ds(off, TBL_SLICE)],
                          avb.at[pl.ds(0, TBL_SLICE)], sem_a).wait()
    pltpu.make_async_copy(sr_hbm.at[pl.ds(off, TBL_SLICE)],
                          bvb.at[pl.ds(0, TBL_SLICE)], sem_a).wait()
    pltpu.async_copy(avb.at[pl.ds(0, TBL_SLICE)],
                     sl_sh.at[pl.ds(off, TBL_SLICE)], sem_a)
    pltpu.async_copy(bvb.at[pl.ds(0, TBL_SLICE)],
                     sr_sh.at[pl.ds(off, TBL_SLICE)], sem_a)

    zero = jnp.zeros((L,), jnp.float32)

    def zbody(i, _):
        accd[pl.ds(i * L, L)] = zero
        accn[pl.ds(i * L, L)] = zero
        return 0

    lax.fori_loop(0, ACC // L, zbody, 0)

    pltpu.make_async_copy(src_hbm.at[pl.ds(e0, EPT)],
                          srcb.at[pl.ds(0, EPT)], sem_c).wait()
    pltpu.make_async_copy(dst_hbm.at[pl.ds(e0, EPT)],
                          dstb.at[pl.ds(0, EPT)], sem_c).wait()
    pltpu.make_async_copy(typ_hbm.at[pl.ds(e0, EPT)],
                          typb.at[pl.ds(0, EPT)], sem_c).wait()

    dummy = jnp.full((L,), DUMMY, jnp.int32)
    izero = jnp.zeros((L,), jnp.int32)

    def tbody(i, _):
        k = EPT + i * L
        srcb[pl.ds(k, L)] = dummy
        dstb[pl.ds(k, L)] = izero
        typb[pl.ds(k, L)] = izero
        return 0

    lax.fori_loop(0, (PADD - EPT) // L, tbody, 0)

    def ibody(i, _):
        k = i * L
        t16 = typb[pl.ds(k, L)]
        ilb[pl.ds(k, L)] = srcb[pl.ds(k, L)] * N_REL + t16
        irb[pl.ds(k, L)] = dstb[pl.ds(k, L)] * N_REL + t16
        return 0

    lax.fori_loop(0, PADD // L, ibody, 0)

    pltpu.make_async_copy(avb.at[pl.ds(0, TBL_SLICE)],
                          sl_sh.at[pl.ds(off, TBL_SLICE)], sem_a).wait()
    pltpu.make_async_copy(bvb.at[pl.ds(0, TBL_SLICE)],
                          sr_sh.at[pl.ds(off, TBL_SLICE)], sem_a).wait()
    pltpu.make_async_copy(hat_hbm, hatb, sem_b).wait()
    pltpu.make_async_copy(c_hbm, cb, sem_b).wait()
    cvec = cb[...]

    plsc.subcore_barrier()

    def fire_block(b, sem):
        def fire(r, _):
            k = (b * RPB + r) * 128
            pltpu.async_copy(sl_sh.at[ilb.at[pl.ds(k, 128)]],
                             avb.at[pl.ds(k, 128)], sem)
            pltpu.async_copy(sr_sh.at[irb.at[pl.ds(k, 128)]],
                             bvb.at[pl.ds(k, 128)], sem)
            return 0
        lax.fori_loop(0, RPB, fire, 0)

    def drain_block(b, sem):
        def drain(r, _):
            k = (b * RPB + r) * 128
            pltpu.make_async_copy(sl_sh.at[ilb.at[pl.ds(k, 128)]],
                                  avb.at[pl.ds(k, 128)], sem).wait()
            pltpu.make_async_copy(sr_sh.at[irb.at[pl.ds(k, 128)]],
                                  bvb.at[pl.ds(k, 128)], sem).wait()
            return 0
        lax.fori_loop(0, RPB, drain, 0)

    def compute_block(b):
        def ebody(i, _):
            k = b * RPB * 128 + i * L
            s = avb[pl.ds(k, L)] + bvb[pl.ds(k, L)]
            e = jnp.maximum(s, 0.2 * s)
            ex = jnp.exp(e - cvec)
            h = plsc.load_gather(hatb, [dstb[pl.ds(k, L)]])
            s16 = srcb[pl.ds(k, L)]
            plsc.addupdate_scatter(accd, [s16], ex)
            plsc.addupdate_scatter(accn, [s16], ex * h)
            return 0
        lax.fori_loop(0, RPB * 128 // L, ebody, 0)

    fire_block(0, sem_g0)

    def pipe(bp, _):
        b0 = bp * 2

        @pl.when(b0 + 1 < NB)
        def _():
            fire_block(b0 + 1, sem_g1)

        drain_block(b0, sem_g0)
        compute_block(b0)

        @pl.when(b0 + 2 < NB)
        def _():
            fire_block(b0 + 2, sem_g0)

        @pl.when(b0 + 1 < NB)
        def _():
            drain_block(b0 + 1, sem_g1)
            compute_block(b0 + 1)

        return 0

    lax.fori_loop(0, (NB + 1) // 2, pipe, 0)

    pltpu.sync_copy(accd, parts_hbm.at[pl.ds(wid * 2 * ACC, ACC)])
    pltpu.sync_copy(accn, parts_hbm.at[pl.ds(wid * 2 * ACC + ACC, ACC)])


@functools.cache
def _edge_kernel():
  return functools.partial(
    pl.kernel,
    out_type=jax.ShapeDtypeStruct((NW * 2 * ACC,), jnp.float32),
    mesh=plsc.VectorSubcoreMesh(core_axis_name="c", subcore_axis_name="s",
                                num_cores=NC, num_subcores=NS),
    compiler_params=pltpu.CompilerParams(needs_layout_passes=False),
    scratch_types=[
        pltpu.VMEM((PADD,), jnp.int32),
        pltpu.VMEM((PADD,), jnp.int32),
        pltpu.VMEM((PADD,), jnp.int32),
        pltpu.VMEM((PADD,), jnp.int32),
        pltpu.VMEM((PADD,), jnp.int32),
        pltpu.VMEM((PADD,), jnp.float32),
        pltpu.VMEM((PADD,), jnp.float32),
        pltpu.VMEM((N_UNITS,), jnp.float32),
        pltpu.VMEM((ACC,), jnp.float32),
        pltpu.VMEM((ACC,), jnp.float32),
        pltpu.VMEM((L,), jnp.float32),
        pltpu.VMEM_SHARED((TBL,), jnp.float32),
        pltpu.VMEM_SHARED((TBL,), jnp.float32),
        pltpu.SemaphoreType.DMA,
        pltpu.SemaphoreType.DMA,
        pltpu.SemaphoreType.DMA,
        pltpu.SemaphoreType.DMA,
        pltpu.SemaphoreType.DMA,
    ],
  )(_edge_body)


def _combine_body(p_ref, o_ref):
    tot = jnp.sum(p_ref[...], axis=0, keepdims=True)
    den = tot[:, :ACC]
    num = tot[:, ACC:]
    o_ref[...] = jnp.where(den != 0.0, num / den, 0.0)


def _combine(parts):
    return pl.pallas_call(
        _combine_body,
        out_shape=jax.ShapeDtypeStruct((1, ACC), jnp.float32),
    )(parts)


def kernel(x_unit_base, hat_t, rel_edge_index, rel_edge_type, a_r_params):
    w = jnp.concatenate([a_r_params[:, :NODE_DIM].T,
                         a_r_params[:, NODE_DIM:].T], axis=1)
    s, c = _scores(x_unit_base, w)

    sl_flat = jnp.pad(s[:, :N_REL].reshape(-1), (0, TBL - N_UNITS * N_REL))
    sr_flat = jnp.pad(s[:, N_REL:].reshape(-1), (0, TBL - N_UNITS * N_REL))
    cvec = jnp.broadcast_to(c.reshape(()), (L,))

    src = rel_edge_index[0].astype(jnp.int32)
    dst = rel_edge_index[1].astype(jnp.int32)
    typ = rel_edge_type.astype(jnp.int32)

    parts = _edge_kernel()(sl_flat, sr_flat, hat_t, src, dst, typ, cvec)
    g = _combine(parts.reshape(NW, 2 * ACC))
    return g[0, :N_UNITS, None]


# speedup vs baseline: 121.7531x; 1.2099x over previous
import functools

import jax
import jax.numpy as jnp
from jax import lax
from jax.experimental import pallas as pl
from jax.experimental.pallas import tpu as pltpu
from jax.experimental.pallas import tpu_sc as plsc

NC = 2
NS = 16
NW = NC * NS
L = 16

N_UNITS = 10000
N_EDGES = 320000
NODE_DIM = 128
N_REL = 16

UPAD = 10240
UBLK = UPAD // 4
EPT = N_EDGES // NW
PADD = 10240
ROWS = PADD // 128
NB = 10
RPB = ROWS // NB
ACC = 10240
DUMMY = N_UNITS
TBL = UBLK * 128
TBL_SLICE = TBL // NS
HSLICE = TBL_SLICE // 2


def _scores_body(x_ref, w_ref, y_ref, c_ref):
    s = jnp.dot(x_ref[...], w_ref[...],
                preferred_element_type=jnp.float32,
                precision=lax.Precision.HIGHEST)
    ys = [s[j * UBLK:(j + 1) * UBLK, :] for j in range(4)]
    y_ref[...] = jnp.concatenate(ys, axis=1)
    c = jnp.max(s[:, :N_REL]) + jnp.max(s[:, N_REL:])
    c_ref[0, 0] = jnp.maximum(c, 0.2 * c)


def _scores(x, w):
    return pl.pallas_call(
        _scores_body,
        out_shape=[
            jax.ShapeDtypeStruct((UBLK, 128), jnp.float32),
            jax.ShapeDtypeStruct((1, 1), jnp.float32),
        ],
        out_specs=[
            pl.BlockSpec(memory_space=pltpu.VMEM),
            pl.BlockSpec(memory_space=pltpu.SMEM),
        ],
    )(x, w)


def _edge_body(tbl_hbm, hat_hbm, src_hbm, dst_hbm, typ_hbm, c_hbm,
               parts_hbm,
               srcb, dstb, typb, ilb, irb, avb, bvb, hatb, accd, accn, cb,
               tbl_sh, sem_a, sem_b, sem_c, sem_g0, sem_g1):
    cid = lax.axis_index("c")
    sid = lax.axis_index("s")
    wid = sid * NC + cid
    e0 = wid * EPT

    off = sid * TBL_SLICE
    pltpu.async_copy(tbl_hbm.at[pl.ds(off, HSLICE)], avb, sem_a)
    pltpu.async_copy(tbl_hbm.at[pl.ds(off + HSLICE, HSLICE)], bvb, sem_a)
    pltpu.async_copy(hat_hbm, hatb, sem_b)
    pltpu.async_copy(c_hbm, cb, sem_b)
    pltpu.async_copy(src_hbm.at[pl.ds(e0, EPT)], srcb.at[pl.ds(0, EPT)], sem_c)
    pltpu.async_copy(dst_hbm.at[pl.ds(e0, EPT)], dstb.at[pl.ds(0, EPT)], sem_c)
    pltpu.async_copy(typ_hbm.at[pl.ds(e0, EPT)], typb.at[pl.ds(0, EPT)], sem_c)

    pltpu.make_async_copy(tbl_hbm.at[pl.ds(off, HSLICE)], avb, sem_a).wait()
    pltpu.make_async_copy(tbl_hbm.at[pl.ds(off + HSLICE, HSLICE)], bvb,
                          sem_a).wait()
    pltpu.async_copy(avb, tbl_sh.at[pl.ds(off, HSLICE)], sem_a)
    pltpu.async_copy(bvb, tbl_sh.at[pl.ds(off + HSLICE, HSLICE)], sem_a)

    zero = jnp.zeros((L,), jnp.float32)

    def zbody(i, _):
        accd[pl.ds(i * L, L)] = zero
        accn[pl.ds(i * L, L)] = zero
        return 0

    lax.fori_loop(0, ACC // L, zbody, 0)

    pltpu.make_async_copy(src_hbm.at[pl.ds(e0, EPT)],
                          srcb.at[pl.ds(0, EPT)], sem_c).wait()
    pltpu.make_async_copy(dst_hbm.at[pl.ds(e0, EPT)],
                          dstb.at[pl.ds(0, EPT)], sem_c).wait()
    pltpu.make_async_copy(typ_hbm.at[pl.ds(e0, EPT)],
                          typb.at[pl.ds(0, EPT)], sem_c).wait()

    dummy = jnp.full((L,), DUMMY, jnp.int32)
    izero = jnp.zeros((L,), jnp.int32)

    def tbody(i, _):
        k = EPT + i * L
        srcb[pl.ds(k, L)] = dummy
        dstb[pl.ds(k, L)] = izero
        typb[pl.ds(k, L)] = izero
        return 0

    lax.fori_loop(0, (PADD - EPT) // L, tbody, 0)

    def ibody(i, _):
        k = i * L
        t16 = typb[pl.ds(k, L)]
        s16 = srcb[pl.ds(k, L)]
        d16 = dstb[pl.ds(k, L)]
        one = jnp.ones((L,), jnp.int32)
        zer = jnp.zeros((L,), jnp.int32)
        sq = (jnp.where(s16 >= UBLK, one, zer)
              + jnp.where(s16 >= 2 * UBLK, one, zer)
              + jnp.where(s16 >= 3 * UBLK, one, zer))
        dq = (jnp.where(d16 >= UBLK, one, zer)
              + jnp.where(d16 >= 2 * UBLK, one, zer)
              + jnp.where(d16 >= 3 * UBLK, one, zer))
        ilb[pl.ds(k, L)] = (s16 - sq * UBLK) * 128 + sq * 32 + t16
        irb[pl.ds(k, L)] = (d16 - dq * UBLK) * 128 + dq * 32 + (t16 + N_REL)
        return 0

    lax.fori_loop(0, PADD // L, ibody, 0)

    pltpu.make_async_copy(avb, tbl_sh.at[pl.ds(off, HSLICE)], sem_a).wait()
    pltpu.make_async_copy(bvb, tbl_sh.at[pl.ds(off + HSLICE, HSLICE)],
                          sem_a).wait()
    pltpu.make_async_copy(hat_hbm, hatb, sem_b).wait()
    pltpu.make_async_copy(c_hbm, cb, sem_b).wait()
    cvec = cb[...]

    plsc.subcore_barrier()

    def fire_block(b, sem):
        def fire(r, _):
            k = (b * RPB + r) * 128
            pltpu.async_copy(tbl_sh.at[ilb.at[pl.ds(k, 128)]],
                             avb.at[pl.ds(k, 128)], sem)
            pltpu.async_copy(tbl_sh.at[irb.at[pl.ds(k, 128)]],
                             bvb.at[pl.ds(k, 128)], sem)
            return 0
        lax.fori_loop(0, RPB, fire, 0)

    def drain_block(b, sem):
        def drain(r, _):
            k = (b * RPB + r) * 128
            pltpu.make_async_copy(tbl_sh.at[ilb.at[pl.ds(k, 128)]],
                                  avb.at[pl.ds(k, 128)], sem).wait()
            pltpu.make_async_copy(tbl_sh.at[irb.at[pl.ds(k, 128)]],
                                  bvb.at[pl.ds(k, 128)], sem).wait()
            return 0
        lax.fori_loop(0, RPB, drain, 0)

    def compute_block(b):
        def ebody(i, _):
            k = b * RPB * 128 + i * L
            s = avb[pl.ds(k, L)] + bvb[pl.ds(k, L)]
            e = jnp.maximum(s, 0.2 * s)
            ex = jnp.exp(e - cvec)
            h = plsc.load_gather(hatb, [dstb[pl.ds(k, L)]])
            s16 = srcb[pl.ds(k, L)]
            plsc.addupdate_scatter(accd, [s16], ex)
            plsc.addupdate_scatter(accn, [s16], ex * h)
            return 0
        lax.fori_loop(0, RPB * 128 // L, ebody, 0)

    fire_block(0, sem_g0)

    def pipe(bp, _):
        b0 = bp * 2

        @pl.when(b0 + 1 < NB)
        def _():
            fire_block(b0 + 1, sem_g1)

        drain_block(b0, sem_g0)
        compute_block(b0)

        @pl.when(b0 + 2 < NB)
        def _():
            fire_block(b0 + 2, sem_g0)

        @pl.when(b0 + 1 < NB)
        def _():
            drain_block(b0 + 1, sem_g1)
            compute_block(b0 + 1)

        return 0

    lax.fori_loop(0, (NB + 1) // 2, pipe, 0)

    pltpu.sync_copy(accd, parts_hbm.at[pl.ds(wid * 2 * ACC, ACC)])
    pltpu.sync_copy(accn, parts_hbm.at[pl.ds(wid * 2 * ACC + ACC, ACC)])


@functools.cache
def _edge_kernel():
  return functools.partial(
    pl.kernel,
    out_type=jax.ShapeDtypeStruct((NW * 2 * ACC,), jnp.float32),
    mesh=plsc.VectorSubcoreMesh(core_axis_name="c", subcore_axis_name="s",
                                num_cores=NC, num_subcores=NS),
    compiler_params=pltpu.CompilerParams(needs_layout_passes=False),
    scratch_types=[
        pltpu.VMEM((PADD,), jnp.int32),
        pltpu.VMEM((PADD,), jnp.int32),
        pltpu.VMEM((PADD,), jnp.int32),
        pltpu.VMEM((PADD,), jnp.int32),
        pltpu.VMEM((PADD,), jnp.int32),
        pltpu.VMEM((PADD,), jnp.float32),
        pltpu.VMEM((PADD,), jnp.float32),
        pltpu.VMEM((N_UNITS,), jnp.float32),
        pltpu.VMEM((ACC,), jnp.float32),
        pltpu.VMEM((ACC,), jnp.float32),
        pltpu.VMEM((L,), jnp.float32),
        pltpu.VMEM_SHARED((TBL,), jnp.float32),
        pltpu.SemaphoreType.DMA,
        pltpu.SemaphoreType.DMA,
        pltpu.SemaphoreType.DMA,
        pltpu.SemaphoreType.DMA,
        pltpu.SemaphoreType.DMA,
    ],
  )(_edge_body)


def _combine_body(p_ref, o_ref):
    den = p_ref[pl.ds(0, ROWS), :]
    num = p_ref[pl.ds(ROWS, ROWS), :]
    for w in range(1, NW):
        den = den + p_ref[pl.ds(w * 2 * ROWS, ROWS), :]
        num = num + p_ref[pl.ds(w * 2 * ROWS + ROWS, ROWS), :]
    o_ref[...] = jnp.where(den != 0.0, num / den, 0.0)


def _combine(parts):
    return pl.pallas_call(
        _combine_body,
        out_shape=jax.ShapeDtypeStruct((ROWS, 128), jnp.float32),
    )(parts)


def kernel(x_unit_base, hat_t, rel_edge_index, rel_edge_type, a_r_params):
    w = jnp.concatenate([a_r_params[:, :NODE_DIM].T,
                         a_r_params[:, NODE_DIM:].T], axis=1)
    xp = jnp.pad(x_unit_base, ((0, UPAD - N_UNITS), (0, 0)))
    y, c = _scores(xp, w)
    tbl_flat = y.reshape(-1)
    cvec = jnp.broadcast_to(c.reshape(()), (L,))

    src = rel_edge_index[0].astype(jnp.int32)
    dst = rel_edge_index[1].astype(jnp.int32)
    typ = rel_edge_type.astype(jnp.int32)

    parts = _edge_kernel()(tbl_flat, hat_t, src, dst, typ, cvec)
    g = _combine(parts.reshape(NW * 2 * ROWS, 128))
    return g.reshape(-1)[:N_UNITS, None]


# speedup vs baseline: 131.5928x; 1.0808x over previous
import functools

import jax
import jax.numpy as jnp
from jax import lax
from jax.experimental import pallas as pl
from jax.experimental.pallas import tpu as pltpu
from jax.experimental.pallas import tpu_sc as plsc

NC = 2
NS = 16
NW = NC * NS
L = 16

N_UNITS = 10000
N_EDGES = 320000
NODE_DIM = 128
N_REL = 16

UPAD = 10240
UBLK = UPAD // 4
EPT = 10112
EPT_LAST = N_EDGES - (NW - 1) * EPT
PADD = 10240
ROWS = PADD // 128
NB = 10
RPB = ROWS // NB
ACC = 10240
DUMMY = N_UNITS
TBL = UBLK * 128
TBL_SLICE = TBL // NS
HSLICE = TBL_SLICE // 2


_XCH = (UBLK, UBLK, UBLK, N_UNITS - 3 * UBLK)


def _scores_body(x_hbm, w_ref, y_ref, c_ref, xb0, xb1, sem0, sem1):
    w = w_ref[...]
    xb = (xb0, xb1)
    sems = (sem0, sem1)

    def copy(j):
        return pltpu.make_async_copy(
            x_hbm.at[pl.ds(j * UBLK, _XCH[j]), :],
            xb[j % 2].at[pl.ds(0, _XCH[j]), :], sems[j % 2])

    copy(0).start()
    copy(1).start()
    ys, mls, mrs = [], [], []
    for j in range(4):
        copy(j).wait()
        yj = jnp.dot(xb[j % 2][pl.ds(0, _XCH[j]), :], w,
                     preferred_element_type=jnp.float32,
                     precision=lax.Precision.HIGHEST)
        if j + 2 < 4:
            copy(j + 2).start()
        mls.append(jnp.max(yj[:, :N_REL]))
        mrs.append(jnp.max(yj[:, N_REL:]))
        if _XCH[j] < UBLK:
            yj = jnp.concatenate(
                [yj, jnp.zeros((UBLK - _XCH[j], 2 * N_REL), jnp.float32)],
                axis=0)
        ys.append(yj)
    y_ref[...] = jnp.concatenate(ys, axis=1)
    c = (jnp.maximum(jnp.maximum(mls[0], mls[1]), jnp.maximum(mls[2], mls[3]))
         + jnp.maximum(jnp.maximum(mrs[0], mrs[1]),
                       jnp.maximum(mrs[2], mrs[3])))
    c_ref[0, 0] = jnp.maximum(c, 0.2 * c)


def _scores(x, w):
    return pl.pallas_call(
        _scores_body,
        out_shape=[
            jax.ShapeDtypeStruct((UBLK, 128), jnp.float32),
            jax.ShapeDtypeStruct((1, 1), jnp.float32),
        ],
        in_specs=[
            pl.BlockSpec(memory_space=pltpu.MemorySpace.HBM),
            pl.BlockSpec(memory_space=pltpu.VMEM),
        ],
        out_specs=[
            pl.BlockSpec(memory_space=pltpu.VMEM),
            pl.BlockSpec(memory_space=pltpu.SMEM),
        ],
        scratch_shapes=[
            pltpu.VMEM((UBLK, 128), jnp.float32),
            pltpu.VMEM((UBLK, 128), jnp.float32),
            pltpu.SemaphoreType.DMA,
            pltpu.SemaphoreType.DMA,
        ],
    )(x, w)


def _edge_body(tbl_hbm, hat_hbm, ei_hbm, typ_hbm, c_hbm,
               parts_hbm,
               edgeb, typb, ilb, irb, avb, bvb, hatb, accd, accn, cb,
               tbl_sh, sem_a, sem_b, sem_c, sem_g0, sem_g1):
    cid = lax.axis_index("c")
    sid = lax.axis_index("s")
    wid = sid * NC + cid
    e0 = wid * EPT
    last = wid == NW - 1

    off = sid * TBL_SLICE
    pltpu.async_copy(tbl_hbm.at[pl.ds(off, HSLICE)], avb, sem_a)
    pltpu.async_copy(tbl_hbm.at[pl.ds(off + HSLICE, HSLICE)], bvb, sem_a)
    pltpu.async_copy(hat_hbm, hatb, sem_b)
    pltpu.async_copy(c_hbm, cb, sem_b)

    @pl.when(jnp.logical_not(last))
    def _():
        pltpu.async_copy(ei_hbm.at[:, pl.ds(e0, EPT)],
                         edgeb.at[:, pl.ds(0, EPT)], sem_c)
        pltpu.async_copy(typ_hbm.at[pl.ds(e0, EPT)],
                         typb.at[pl.ds(0, EPT)], sem_c)

    @pl.when(last)
    def _():
        pltpu.async_copy(ei_hbm.at[:, pl.ds(e0, EPT_LAST)],
                         edgeb.at[:, pl.ds(0, EPT_LAST)], sem_c)
        pltpu.async_copy(typ_hbm.at[pl.ds(e0, EPT_LAST)],
                         typb.at[pl.ds(0, EPT_LAST)], sem_c)

    pltpu.make_async_copy(tbl_hbm.at[pl.ds(off, HSLICE)], avb, sem_a).wait()
    pltpu.make_async_copy(tbl_hbm.at[pl.ds(off + HSLICE, HSLICE)], bvb,
                          sem_a).wait()
    pltpu.async_copy(avb, tbl_sh.at[pl.ds(off, HSLICE)], sem_a)
    pltpu.async_copy(bvb, tbl_sh.at[pl.ds(off + HSLICE, HSLICE)], sem_a)

    zero = jnp.zeros((L,), jnp.float32)

    def zbody(i, _):
        accd[pl.ds(i * L, L)] = zero
        accn[pl.ds(i * L, L)] = zero
        return 0

    lax.fori_loop(0, ACC // L, zbody, 0)

    @pl.when(jnp.logical_not(last))
    def _():
        pltpu.make_async_copy(ei_hbm.at[:, pl.ds(e0, EPT)],
                              edgeb.at[:, pl.ds(0, EPT)], sem_c).wait()
        pltpu.make_async_copy(typ_hbm.at[pl.ds(e0, EPT)],
                              typb.at[pl.ds(0, EPT)], sem_c).wait()

    @pl.when(last)
    def _():
        pltpu.make_async_copy(ei_hbm.at[:, pl.ds(e0, EPT_LAST)],
                              edgeb.at[:, pl.ds(0, EPT_LAST)], sem_c).wait()
        pltpu.make_async_copy(typ_hbm.at[pl.ds(e0, EPT_LAST)],
                              typb.at[pl.ds(0, EPT_LAST)], sem_c).wait()

    dummy = jnp.full((L,), DUMMY, jnp.int32)
    izero = jnp.zeros((L,), jnp.int32)
    ne = jnp.where(last, EPT_LAST, EPT)

    def tbody(i, _):
        k = i * L
        edgeb[0, pl.ds(k, L)] = dummy
        edgeb[1, pl.ds(k, L)] = izero
        typb[pl.ds(k, L)] = izero
        return 0

    lax.fori_loop(ne // L, PADD // L, tbody, 0)

    def ibody(i, _):
        k = i * L
        t16 = typb[pl.ds(k, L)]
        s16 = edgeb[0, pl.ds(k, L)]
        d16 = edgeb[1, pl.ds(k, L)]
        one = jnp.ones((L,), jnp.int32)
        zer = jnp.zeros((L,), jnp.int32)
        sq = (jnp.where(s16 >= UBLK, one, zer)
              + jnp.where(s16 >= 2 * UBLK, one, zer)
              + jnp.where(s16 >= 3 * UBLK, one, zer))
        dq = (jnp.where(d16 >= UBLK, one, zer)
              + jnp.where(d16 >= 2 * UBLK, one, zer)
              + jnp.where(d16 >= 3 * UBLK, one, zer))
        ilb[pl.ds(k, L)] = (s16 - sq * UBLK) * 128 + sq * 32 + t16
        irb[pl.ds(k, L)] = (d16 - dq * UBLK) * 128 + dq * 32 + (t16 + N_REL)
        return 0

    lax.fori_loop(0, PADD // L, ibody, 0)

    pltpu.make_async_copy(avb, tbl_sh.at[pl.ds(off, HSLICE)], sem_a).wait()
    pltpu.make_async_copy(bvb, tbl_sh.at[pl.ds(off + HSLICE, HSLICE)],
                          sem_a).wait()
    pltpu.make_async_copy(hat_hbm, hatb, sem_b).wait()
    pltpu.make_async_copy(c_hbm, cb, sem_b).wait()
    cvec = cb[...]

    plsc.subcore_barrier()

    def fire_block(b, sem):
        def fire(r, _):
            k = (b * RPB + r) * 128
            pltpu.async_copy(tbl_sh.at[ilb.at[pl.ds(k, 128)]],
                             avb.at[pl.ds(k, 128)], sem)
            pltpu.async_copy(tbl_sh.at[irb.at[pl.ds(k, 128)]],
                             bvb.at[pl.ds(k, 128)], sem)
            return 0
        lax.fori_loop(0, RPB, fire, 0)

    def drain_block(b, sem):
        def drain(r, _):
            k = (b * RPB + r) * 128
            pltpu.make_async_copy(tbl_sh.at[ilb.at[pl.ds(k, 128)]],
                                  avb.at[pl.ds(k, 128)], sem).wait()
            pltpu.make_async_copy(tbl_sh.at[irb.at[pl.ds(k, 128)]],
                                  bvb.at[pl.ds(k, 128)], sem).wait()
            return 0
        lax.fori_loop(0, RPB, drain, 0)

    def compute_block(b):
        def ebody(i, _):
            k = b * RPB * 128 + i * L
            s = avb[pl.ds(k, L)] + bvb[pl.ds(k, L)]
            e = jnp.maximum(s, 0.2 * s)
            ex = jnp.exp(e - cvec)
            h = plsc.load_gather(hatb, [edgeb[1, pl.ds(k, L)]])
            s16 = edgeb[0, pl.ds(k, L)]
            plsc.addupdate_scatter(accd, [s16], ex)
            plsc.addupdate_scatter(accn, [s16], ex * h)
            return 0
        lax.fori_loop(0, RPB * 128 // L, ebody, 0)

    fire_block(0, sem_g0)

    def pipe(bp, _):
        b0 = bp * 2

        @pl.when(b0 + 1 < NB)
        def _():
            fire_block(b0 + 1, sem_g1)

        drain_block(b0, sem_g0)
        compute_block(b0)

        @pl.when(b0 + 2 < NB)
        def _():
            fire_block(b0 + 2, sem_g0)

        @pl.when(b0 + 1 < NB)
        def _():
            drain_block(b0 + 1, sem_g1)
            compute_block(b0 + 1)

        return 0

    lax.fori_loop(0, (NB + 1) // 2, pipe, 0)

    pltpu.sync_copy(accd, parts_hbm.at[pl.ds(wid * 2 * ACC, ACC)])
    pltpu.sync_copy(accn, parts_hbm.at[pl.ds(wid * 2 * ACC + ACC, ACC)])


@functools.cache
def _edge_kernel():
  return functools.partial(
    pl.kernel,
    out_type=jax.ShapeDtypeStruct((NW * 2 * ACC,), jnp.float32),
    mesh=plsc.VectorSubcoreMesh(core_axis_name="c", subcore_axis_name="s",
                                num_cores=NC, num_subcores=NS),
    compiler_params=pltpu.CompilerParams(needs_layout_passes=False),
    scratch_types=[
        pltpu.VMEM((2, PADD), jnp.int32),
        pltpu.VMEM((PADD,), jnp.int32),
        pltpu.VMEM((PADD,), jnp.int32),
        pltpu.VMEM((PADD,), jnp.int32),
        pltpu.VMEM((PADD,), jnp.float32),
        pltpu.VMEM((PADD,), jnp.float32),
        pltpu.VMEM((N_UNITS,), jnp.float32),
        pltpu.VMEM((ACC,), jnp.float32),
        pltpu.VMEM((ACC,), jnp.float32),
        pltpu.VMEM((L,), jnp.float32),
        pltpu.VMEM_SHARED((TBL,), jnp.float32),
        pltpu.SemaphoreType.DMA,
        pltpu.SemaphoreType.DMA,
        pltpu.SemaphoreType.DMA,
        pltpu.SemaphoreType.DMA,
        pltpu.SemaphoreType.DMA,
    ],
  )(_edge_body)


def _combine_body(p_ref, o_ref):
    den = p_ref[pl.ds(0, ROWS), :]
    num = p_ref[pl.ds(ROWS, ROWS), :]
    for w in range(1, NW):
        den = den + p_ref[pl.ds(w * 2 * ROWS, ROWS), :]
        num = num + p_ref[pl.ds(w * 2 * ROWS + ROWS, ROWS), :]
    o_ref[...] = jnp.where(den != 0.0, num / den, 0.0)


def _combine(parts):
    return pl.pallas_call(
        _combine_body,
        out_shape=jax.ShapeDtypeStruct((ROWS, 128), jnp.float32),
    )(parts)


def kernel(x_unit_base, hat_t, rel_edge_index, rel_edge_type, a_r_params):
    w = jnp.concatenate([a_r_params[:, :NODE_DIM].T,
                         a_r_params[:, NODE_DIM:].T], axis=1)
    y, c = _scores(x_unit_base, w)
    tbl_flat = y.reshape(-1)
    cvec = jnp.broadcast_to(c.reshape(()), (L,))

    parts = _edge_kernel()(tbl_flat, hat_t, rel_edge_index, rel_edge_type,
                           cvec)
    g = _combine(parts.reshape(NW * 2 * ROWS, 128))
    return g.reshape(-1)[:N_UNITS, None]
